# baseline (speedup 1.0000x reference)
import functools

import jax
import jax.numpy as jnp
from jax import lax
from jax.experimental import pallas as pl
from jax.experimental.pallas import tpu as pltpu
from jax.experimental.pallas import tpu_sc as plsc


_NC = 2
_NS = 16
_NW = _NC * _NS
_LANES = 16
_RB = 16


def _sc_body(x_hbm, pos_hbm, out_hbm, pos_buf, x_buf, o_buf):
    b, sd = x_hbm.shape
    d = 768
    rows = sd // d
    chunk = rows // _NW
    nblk = chunk // _RB
    blk_elems = _RB * d

    wid = lax.axis_index("s") * _NC + lax.axis_index("c")
    base = wid * chunk * d

    def blk_loop(j, carry):
        off = base + j * blk_elems
        pltpu.sync_copy(pos_hbm.at[pl.ds(off, blk_elems)], pos_buf)
        for bi in range(b):
            pltpu.sync_copy(x_hbm.at[bi, pl.ds(off, blk_elems)], x_buf)

            def add_row(i, c):
                r = i * d
                for k in range(d // _LANES):
                    sl = pl.ds(r + k * _LANES, _LANES)
                    o_buf[sl] = x_buf[sl] + pos_buf[sl]
                return c

            lax.fori_loop(0, _RB, add_row, 0)
            pltpu.sync_copy(o_buf, out_hbm.at[bi, pl.ds(off, blk_elems)])
        return carry

    lax.fori_loop(0, nblk, blk_loop, 0)


def kernel(x, pos_embedding):
    b, s, d = x.shape
    x2 = x.reshape(b, s * d)
    pos = pos_embedding.reshape(s * d)
    blk_elems = _RB * d

    mesh = plsc.VectorSubcoreMesh(core_axis_name="c", subcore_axis_name="s")
    run = functools.partial(
        pl.kernel,
        mesh=mesh,
        out_type=jax.ShapeDtypeStruct((b, s * d), jnp.float32),
        scratch_types=[
            pltpu.VMEM((blk_elems,), jnp.float32),
            pltpu.VMEM((blk_elems,), jnp.float32),
            pltpu.VMEM((blk_elems,), jnp.float32),
        ],
    )(_sc_body)
    out = run(x2, pos)
    return out.reshape(b, s, d)


# speedup vs baseline: 5.2897x; 5.2897x over previous
import jax
import jax.numpy as jnp
from jax.experimental import pallas as pl


_SEQ_BLK = 1024


def _body(x_ref, pos_ref, o_ref):
    o_ref[...] = x_ref[...] + pos_ref[...][None, :, :]


def kernel(x, pos_embedding):
    b, s, d = x.shape
    blk = _SEQ_BLK
    if s % blk != 0:
        blk = s
    grid = (s // blk,)
    return pl.pallas_call(
        _body,
        grid=grid,
        in_specs=[
            pl.BlockSpec((b, blk, d), lambda i: (0, i, 0)),
            pl.BlockSpec((blk, d), lambda i: (i, 0)),
        ],
        out_specs=pl.BlockSpec((b, blk, d), lambda i: (0, i, 0)),
        out_shape=jax.ShapeDtypeStruct(x.shape, x.dtype),
    )(x, pos_embedding)
